# baseline (speedup 1.0000x reference)
import jax
import jax.numpy as jnp
from jax import lax
from jax.experimental import pallas as pl
from jax.experimental.pallas import tpu as pltpu
from jax.experimental.pallas import tpu_sc as plsc

_L = 16
_NC = 2
_NS = 16
_RPW = 8
_D = 64
_NBUF = 4


def _sc_body(embT, neg, ctx, src, out, negv, ctxv, srcv, blocks_v, rows_v,
             srows_v, part_v, comb_v, out_v, shared, sem):
    cid = lax.axis_index("c")
    s = lax.axis_index("s")
    wid = s * _NC + cid
    base = pl.multiple_of(wid * _RPW, _RPW)

    pltpu.sync_copy(neg, negv)
    pltpu.sync_copy(ctx, ctxv)
    pltpu.sync_copy(src, srcv)

    lanes = lax.iota(jnp.int32, _L)
    lanes8 = jnp.bitwise_and(lanes, _RPW - 1)
    g = lanes8 + base
    n_i = plsc.load_gather(negv, [jnp.minimum(g, 199)])
    c_i = plsc.load_gather(ctxv, [jnp.clip(g - 200, 0, 49)])
    s_i = plsc.load_gather(srcv, [jnp.zeros((_L,), jnp.int32)])
    vidx = jnp.where(g < 200, n_i, jnp.where(g < 250, c_i, s_i))

    nrows = _RPW + 1
    rs = [vidx[i] for i in range(_RPW)] + [s_i[0]]

    def issue(k):
        blk = pl.multiple_of(
            lax.shift_left(lax.shift_right_logical(rs[k], 7), 7), 128)
        return pltpu.async_copy(
            embT.at[:, pl.ds(blk, 128)], blocks_v.at[k % _NBUF], sem)

    def drain(k, cp):
        cp.wait()
        lanevec = jnp.zeros((_L,), jnp.int32) + jnp.bitwise_and(rs[k], 127)
        for c in range(_D // _L):
            chunk = plsc.load_gather(
                blocks_v.at[k % _NBUF], [c * _L + lanes, lanevec])
            if k < _RPW:
                rows_v[k, pl.ds(c * _L, _L)] = chunk
            else:
                srows_v[0, pl.ds(c * _L, _L)] = chunk

    pend = {}
    for k in range(nrows):
        if k >= _NBUF:
            drain(k - _NBUF, pend.pop(k - _NBUF))
        pend[k] = issue(k)
    for k in range(nrows - _NBUF, nrows):
        drain(k, pend.pop(k))

    def _bf16r(x):
        b = plsc.bitcast(x, jnp.int32)
        r = b + 0x7FFF + jnp.bitwise_and(lax.shift_right_logical(b, 16), 1)
        return plsc.bitcast(jnp.bitwise_and(r, jnp.int32(-65536)), jnp.float32)

    acc = jnp.zeros((_L,), jnp.float32)
    src_chunks = [_bf16r(srows_v[0, pl.ds(c * _L, _L)])
                  for c in range(_D // _L)]
    for d in range(_D):
        col = plsc.load_gather(rows_v, [lanes8, jnp.full((_L,), d, jnp.int32)])
        acc = acc + _bf16r(col) * src_chunks[d // _L][d % _L]

    live = lanes < _RPW
    sig = 1.0 / (1.0 + jnp.exp(acc))
    part_v[0, :] = jnp.where(jnp.logical_and(live, g < 200), sig, 0.0)
    part_v[1, :] = jnp.where(
        jnp.logical_and(live, jnp.logical_and(g >= 200, g < 250)), acc, 0.0)
    pltpu.sync_copy(part_v, shared.at[pl.ds(2 * s, 2)])
    plsc.subcore_barrier()

    @pl.when(s == 0)
    def _():
        pltpu.sync_copy(shared, comb_v)
        nacc = jnp.zeros((_L,), jnp.float32)
        pacc = jnp.zeros((_L,), jnp.float32)
        for i in range(_NS):
            nacc = nacc + comb_v[2 * i, :]
            pacc = pacc + comb_v[2 * i + 1, :]
        out_v[0, :] = jnp.zeros((_L,), jnp.float32) + jnp.sum(nacc)
        out_v[1, :] = jnp.zeros((_L,), jnp.float32) + jnp.sum(pacc)
        pltpu.sync_copy(out_v, out.at[cid])


@jax.jit
def _sc_parts(embT, neg, ctx, src):
    f = pl.kernel(
        _sc_body,
        out_type=jax.ShapeDtypeStruct((_NC, 2, _L), jnp.float32),
        mesh=plsc.VectorSubcoreMesh(
            core_axis_name="c", subcore_axis_name="s",
            num_cores=_NC, num_subcores=_NS),
        scratch_types=[
            pltpu.VMEM((200,), jnp.int32),
            pltpu.VMEM((50,), jnp.int32),
            pltpu.VMEM((1,), jnp.int32),
            pltpu.VMEM((_NBUF, _D, 128), jnp.float32),
            pltpu.VMEM((_RPW, _D), jnp.float32),
            pltpu.VMEM((1, _D), jnp.float32),
            pltpu.VMEM((2, _L), jnp.float32),
            pltpu.VMEM((2 * _NS, _L), jnp.float32),
            pltpu.VMEM((2, _L), jnp.float32),
            pltpu.VMEM_SHARED((2 * _NS, _L), jnp.float32),
            pltpu.SemaphoreType.DMA,
        ],
        compiler_params=pltpu.CompilerParams(needs_layout_passes=False),
    )
    return f(embT, neg, ctx, src)


def kernel(embedding, source_node, context_nodes, neg_samples):
    parts = _sc_parts(
        embedding.T,
        neg_samples.astype(jnp.int32),
        context_nodes.astype(jnp.int32),
        source_node.astype(jnp.int32),
    )
    nsum = parts[0, 0, 0] + parts[1, 0, 0]
    psum = parts[0, 1, 0] + parts[1, 1, 0]
    positives = jnp.clip(jax.nn.sigmoid(psum), 1e-7, 1 - 1e-7)
    negatives = jnp.clip(nsum, 1e-7, 1 - 1e-7)
    return -jnp.log(positives) - negatives


# speedup vs baseline: 1.1804x; 1.1804x over previous
import jax
import jax.numpy as jnp
from jax import lax
from jax.experimental import pallas as pl
from jax.experimental.pallas import tpu as pltpu
from jax.experimental.pallas import tpu_sc as plsc

_L = 16
_NS = 16
_D = 64
_NBUF = 4
_LN2 = 0.6931471805599453


def _sc_body(embT, neg, ctx, src, out, negv, ctxv, srcv, blocks_v, rows_v,
             srows_v, part_v, comb_v, out_v, shared, sem):
    w = lax.axis_index("s")
    base = pl.multiple_of(w * _L, _L)

    pltpu.sync_copy(neg, negv)
    pltpu.sync_copy(ctx, ctxv)
    pltpu.sync_copy(src, srcv)

    lanes = lax.iota(jnp.int32, _L)
    g = lanes + base
    n_i = plsc.load_gather(negv, [jnp.minimum(g, 199)])
    c_i = plsc.load_gather(ctxv, [jnp.clip(g - 200, 0, 49)])
    s_i = plsc.load_gather(srcv, [jnp.zeros((_L,), jnp.int32)])
    vidx = jnp.where(g < 200, n_i, jnp.where(g < 250, c_i, s_i))

    nrows = _L + 1
    rs = [vidx[i] for i in range(_L)] + [s_i[0]]

    def issue(k):
        blk = pl.multiple_of(
            lax.shift_left(lax.shift_right_logical(rs[k], 7), 7), 128)
        return pltpu.async_copy(
            embT.at[:, pl.ds(blk, 128)], blocks_v.at[k % _NBUF], sem)

    def drain(k, cp):
        cp.wait()
        lane = jnp.bitwise_and(rs[k], 127)
        lanevec = jnp.zeros((_L,), jnp.int32) + lane
        for c in range(_D // _L):
            chunk = plsc.load_gather(
                blocks_v.at[k % _NBUF], [c * _L + lanes, lanevec])
            if k < _L:
                rows_v[k, pl.ds(c * _L, _L)] = chunk
            else:
                srows_v[0, pl.ds(c * _L, _L)] = chunk

    pend = {}
    for k in range(nrows):
        if k >= _NBUF:
            drain(k - _NBUF, pend.pop(k - _NBUF))
        pend[k] = issue(k)
    for k in range(nrows - _NBUF, nrows):
        drain(k, pend.pop(k))

    def _bf16r(x):
        b = plsc.bitcast(x, jnp.int32)
        r = b + 0x7FFF + jnp.bitwise_and(lax.shift_right_logical(b, 16), 1)
        return plsc.bitcast(jnp.bitwise_and(r, jnp.int32(-65536)), jnp.float32)

    acc = jnp.zeros((_L,), jnp.float32)
    src_chunks = [_bf16r(srows_v[0, pl.ds(c * _L, _L)])
                  for c in range(_D // _L)]
    for d in range(_D):
        col = plsc.load_gather(rows_v, [lanes, jnp.full((_L,), d, jnp.int32)])
        acc = acc + _bf16r(col) * src_chunks[d // _L][d % _L]

    sig = 1.0 / (1.0 + jnp.exp(acc))
    part_v[0, :] = jnp.where(g < 200, sig, 0.0)
    part_v[1, :] = jnp.where(jnp.logical_and(g >= 200, g < 250), acc, 0.0)
    pltpu.sync_copy(part_v, shared.at[pl.ds(2 * w, 2)])
    plsc.subcore_barrier()

    @pl.when(w == 0)
    def _():
        pltpu.sync_copy(shared, comb_v)
        nacc = jnp.zeros((_L,), jnp.float32)
        pacc = jnp.zeros((_L,), jnp.float32)
        for i in range(_NS):
            nacc = nacc + comb_v[2 * i, :]
            pacc = pacc + comb_v[2 * i + 1, :]
        nsum = jnp.zeros((_L,), jnp.float32) + jnp.sum(nacc)
        psum = jnp.zeros((_L,), jnp.float32) + jnp.sum(pacc)
        pos = 1.0 / (1.0 + jnp.exp(-psum))
        posc = jnp.clip(pos, 1e-7, 1.0 - 1e-7)
        negc = jnp.clip(nsum, 1e-7, 1.0 - 1e-7)
        bits = plsc.bitcast(posc, jnp.int32)
        e = lax.shift_right_logical(bits, 23) - 127
        m = plsc.bitcast(
            jnp.bitwise_or(jnp.bitwise_and(bits, 0x007FFFFF), 0x3F800000),
            jnp.float32)
        z = (m - 1.0) / (m + 1.0)
        z2 = z * z
        lnm = 2.0 * z * (1.0 + z2 * (1.0 / 3.0 + z2 * (
            0.2 + z2 * (1.0 / 7.0 + z2 * (1.0 / 9.0)))))
        lnp = e.astype(jnp.float32) * _LN2 + lnm
        out_v[...] = -lnp - negc
        pltpu.sync_copy(out_v.at[pl.ds(0, 8)], out)


@jax.jit
def _sc_loss(embT, neg, ctx, src):
    f = pl.kernel(
        _sc_body,
        out_type=jax.ShapeDtypeStruct((8,), jnp.float32),
        mesh=plsc.VectorSubcoreMesh(
            core_axis_name="c", subcore_axis_name="s",
            num_cores=1, num_subcores=_NS),
        scratch_types=[
            pltpu.VMEM((200,), jnp.int32),
            pltpu.VMEM((50,), jnp.int32),
            pltpu.VMEM((1,), jnp.int32),
            pltpu.VMEM((_NBUF, _D, 128), jnp.float32),
            pltpu.VMEM((_L, _D), jnp.float32),
            pltpu.VMEM((1, _D), jnp.float32),
            pltpu.VMEM((2, _L), jnp.float32),
            pltpu.VMEM((2 * _NS, _L), jnp.float32),
            pltpu.VMEM((_L,), jnp.float32),
            pltpu.VMEM_SHARED((2 * _NS, _L), jnp.float32),
            pltpu.SemaphoreType.DMA,
        ],
        compiler_params=pltpu.CompilerParams(needs_layout_passes=False),
    )
    return f(embT, neg, ctx, src)


def kernel(embedding, source_node, context_nodes, neg_samples):
    parts = _sc_loss(
        embedding.T,
        neg_samples.astype(jnp.int32),
        context_nodes.astype(jnp.int32),
        source_node.astype(jnp.int32),
    )
    return parts[0]
